# baseline (speedup 1.0000x reference)
import functools

import jax
import jax.numpy as jnp
from jax import lax
from jax.experimental import pallas as pl
from jax.experimental.pallas import tpu as pltpu
from jax.experimental.pallas import tpu_sc as plsc

B = 4
PPER = 12000
P = B * PPER
C = 64
NX, NY = 432, 496
NC, NS, L = 2, 16, 16
NW = NC * NS
CPW = C // NW
WB = NW // B
BX = 8
NBX = NX // BX
NBINS = B * NBX
MAXK = 7
CAP = 384
PCHUNK = 2000
NPCHUNK = PPER // PCHUNK
NYL = NY // L

_mesh = functools.partial(
    plsc.VectorSubcoreMesh,
    core_axis_name="c", subcore_axis_name="s",
    num_cores=NC, num_subcores=NS,
)

_SC_PARAMS = pltpu.CompilerParams(needs_layout_passes=False)


def _sc_bin(coords):
    @functools.partial(
        pl.kernel,
        out_type=jax.ShapeDtypeStruct((NBINS * CAP,), jnp.int32),
        mesh=_mesh(),
        compiler_params=_SC_PARAMS,
        scratch_types=[
            pltpu.VMEM((PCHUNK * 4,), jnp.int32),
        ] + [pltpu.VMEM((CAP,), jnp.int32) for _ in range(MAXK)],
    )
    def k(coords_hbm, plist_hbm, c_v, *rest):
        lsts = rest[:MAXK]
        wid = lax.axis_index("s") * NC + lax.axis_index("c")
        sub = lax.rem(wid, WB)
        bb = wid // WB
        pbase = bb * PPER
        iota = lax.iota(jnp.int32, L)

        def chunk(t, offs):
            base = pbase + t * PCHUNK
            pltpu.sync_copy(coords_hbm.at[pl.ds(base * 4, PCHUNK * 4)], c_v)

            def vbody(i, offs):
                row = (i * L + iota) * 4
                yy = plsc.load_gather(c_v, [row + 2])
                xx = plsc.load_gather(c_v, [row + 3])
                bx = xx >> 3
                packed = (base + i * L + iota) | ((xx & 7) << 16) | (yy << 19)
                new = []
                for kk in range(MAXK):
                    m = bx == (sub + 8 * kk)
                    plsc.store_compressed(
                        lsts[kk].at[pl.ds(offs[kk], L)], packed, mask=m)
                    new.append(offs[kk] + jnp.sum(m.astype(jnp.int32)))
                return tuple(new)

            return lax.fori_loop(0, PCHUNK // L, vbody, offs)

        offs = tuple(jnp.int32(16) for _ in range(MAXK))
        for t in range(NPCHUNK):
            offs = chunk(t, offs)

        for kk in range(MAXK):
            jb = sub + 8 * kk
            j = bb * NBX + jb

            @pl.when(jb < NBX)
            def _(kk=kk, j=j):
                lsts[kk][pl.ds(0, L)] = jnp.where(iota == 0, offs[kk] - 16, 0)
                pltpu.sync_copy(lsts[kk], plist_hbm.at[pl.ds(j * CAP, CAP)])

    return k(coords)


def _sc_scatter(ft, plist):
    @functools.partial(
        pl.kernel,
        out_type=jax.ShapeDtypeStruct((B, C, NX, NY), jnp.float32),
        mesh=_mesh(),
        compiler_params=_SC_PARAMS,
        scratch_types=[
            pltpu.VMEM((CPW * P,), jnp.float32),
            pltpu.VMEM((2, CPW, BX, NY), jnp.float32),
            pltpu.VMEM((3, CAP), jnp.int32),
            pltpu.SemaphoreType.DMA,
            pltpu.SemaphoreType.DMA,
        ],
    )
    def k(ft_hbm, plist_hbm, out_hbm, r01, st, lst_v, sem0, sem_l):
        wid = lax.axis_index("s") * NC + lax.axis_index("c")
        c0 = wid * CPW
        pltpu.sync_copy(ft_hbm.at[c0], r01.at[pl.ds(0, P)])
        pltpu.sync_copy(ft_hbm.at[c0 + 1], r01.at[pl.ds(P, P)])
        iota = lax.iota(jnp.int32, L)
        zero16 = jnp.zeros((L,), jnp.float32)
        zero16i = jnp.zeros((L,), jnp.int32)
        lst_v[0, pl.ds(0, L)] = zero16i
        lst_v[1, pl.ds(0, L)] = zero16i
        lst_v[2, pl.ds(0, L)] = zero16i

        for p in range(2):
            for kk in range(CPW):
                for xr in range(BX):
                    @plsc.parallel_loop(0, NYL, 1, unroll=8)
                    def _(yi, p=p, kk=kk, xr=xr):
                        st[p, kk, xr, pl.ds(yi * L, L)] = zero16

        def unpack(pc):
            return pc & 0xFFFF, (pc >> 16) & 7, pc >> 19

        def dummy_copy():
            return pltpu.make_async_copy(
                st.at[0], out_hbm.at[0, pl.ds(0, CPW), pl.ds(0, BX), :], sem0)

        def dummy_list_copy():
            return pltpu.make_async_copy(
                plist_hbm.at[pl.ds(0, CAP)], lst_v.at[0], sem_l)

        pltpu.async_copy(plist_hbm.at[pl.ds(0, CAP)], lst_v.at[0], sem_l)

        for b in range(B):
            def task(jb, carry):
                t = b * NBX + jb
                p = lax.rem(t, 2)
                s3 = lax.rem(t, 3)
                u3 = lax.rem(t + 1, 3)
                j = t
                cond = jb >= (2 if b == 0 else 0)

                @pl.when(cond)
                def _():
                    dummy_copy().wait()

                pn = lst_v[u3, pl.ds(0, L)][0]
                pnv = (pn + (L - 1)) >> 4

                def undo(v, carry2):
                    m = iota < (pn - v * L)
                    _, cxl, yy = unpack(lst_v[u3, pl.ds(16 + v * L, L)])
                    plsc.store_scatter(st.at[p, 0], [cxl, yy], zero16, mask=m)
                    plsc.store_scatter(st.at[p, 1], [cxl, yy], zero16, mask=m)
                    return carry2

                lax.fori_loop(0, pnv, undo, 0)

                @pl.when(j + 1 < NBINS)
                def _():
                    pltpu.async_copy(
                        plist_hbm.at[pl.ds((j + 1) * CAP, CAP)],
                        lst_v.at[u3], sem_l)

                dummy_list_copy().wait()
                n = lst_v[s3, pl.ds(0, L)][0]
                nv = (n + (L - 1)) >> 4

                def fill(v, carry2):
                    m = iota < (n - v * L)
                    pid, cxl, yy = unpack(lst_v[s3, pl.ds(16 + v * L, L)])
                    v0 = plsc.load_gather(r01, [pid], mask=m)
                    plsc.store_scatter(st.at[p, 0], [cxl, yy], v0, mask=m)
                    v1 = plsc.load_gather(r01, [pid + P], mask=m)
                    plsc.store_scatter(st.at[p, 1], [cxl, yy], v1, mask=m)
                    return carry2

                lax.fori_loop(0, nv, fill, 0)

                x0 = jb * BX
                pltpu.async_copy(
                    st.at[p],
                    out_hbm.at[b, pl.ds(c0, CPW), pl.ds(x0, BX), :], sem0)
                return carry

            lax.fori_loop(0, NBX, task, 0)

        for _ in range(2):
            dummy_copy().wait()

    return k(ft, plist)


def kernel(pillar_features, coords, batch_size, input_shape):
    del batch_size, input_shape
    coords = coords.astype(jnp.int32).reshape(P * 4)
    ft = pillar_features.T
    plist = _sc_bin(coords)
    out = _sc_scatter(ft, plist)
    return out.swapaxes(2, 3)


# speedup vs baseline: 1.1653x; 1.1653x over previous
import functools

import jax
import jax.numpy as jnp
from jax import lax
from jax.experimental import pallas as pl
from jax.experimental.pallas import tpu as pltpu
from jax.experimental.pallas import tpu_sc as plsc

B = 4
PPER = 12000
P = B * PPER
C = 64
NX, NY = 432, 496
NC, NS, L = 2, 16, 16
NW = NC * NS
CPW = C // NW
WB = NW // B
BX = 8
NBX = NX // BX
NBINS = B * NBX
MAXK = 7
CAP = 384
PCHUNK = 2000
NPCHUNK = PPER // PCHUNK
NYL = NY // L

_mesh = functools.partial(
    plsc.VectorSubcoreMesh,
    core_axis_name="c", subcore_axis_name="s",
    num_cores=NC, num_subcores=NS,
)

_SC_PARAMS = pltpu.CompilerParams(needs_layout_passes=False)


def _sc_bin(cy, cx):
    @functools.partial(
        pl.kernel,
        out_type=jax.ShapeDtypeStruct((NBINS * CAP,), jnp.int32),
        mesh=_mesh(),
        compiler_params=_SC_PARAMS,
        scratch_types=[
            pltpu.VMEM((PCHUNK,), jnp.int32),
            pltpu.VMEM((PCHUNK,), jnp.int32),
        ] + [pltpu.VMEM((CAP,), jnp.int32) for _ in range(MAXK)],
    )
    def k(cy_hbm, cx_hbm, plist_hbm, y_v, x_v, *rest):
        lsts = rest[:MAXK]
        wid = lax.axis_index("s") * NC + lax.axis_index("c")
        sub = lax.rem(wid, WB)
        bb = wid // WB
        pbase = bb * PPER
        iota = lax.iota(jnp.int32, L)

        def chunk(t, offs):
            base = pbase + t * PCHUNK
            pltpu.sync_copy(cy_hbm.at[pl.ds(base, PCHUNK)], y_v)
            pltpu.sync_copy(cx_hbm.at[pl.ds(base, PCHUNK)], x_v)

            def vbody(i, offs):
                yy = y_v[pl.ds(i * L, L)]
                xx = x_v[pl.ds(i * L, L)]
                bx = xx >> 3
                packed = (base + i * L + iota) | ((xx & 7) << 16) | (yy << 19)
                new = []
                for kk in range(MAXK):
                    m = bx == (sub + 8 * kk)
                    plsc.store_compressed(
                        lsts[kk].at[pl.ds(offs[kk], L)], packed, mask=m)
                    new.append(offs[kk] + jnp.sum(m.astype(jnp.int32)))
                return tuple(new)

            return lax.fori_loop(0, PCHUNK // L, vbody, offs)

        offs = tuple(jnp.int32(16) for _ in range(MAXK))
        for t in range(NPCHUNK):
            offs = chunk(t, offs)

        for kk in range(MAXK):
            jb = sub + 8 * kk
            j = bb * NBX + jb

            @pl.when(jb < NBX)
            def _(kk=kk, j=j):
                lsts[kk][pl.ds(0, L)] = jnp.where(iota == 0, offs[kk] - 16, 0)
                pltpu.sync_copy(lsts[kk], plist_hbm.at[pl.ds(j * CAP, CAP)])

    return k(cy, cx)


def _sc_scatter(ft, plist):
    @functools.partial(
        pl.kernel,
        out_type=jax.ShapeDtypeStruct((B, C, NX, NY), jnp.float32),
        mesh=_mesh(),
        compiler_params=_SC_PARAMS,
        scratch_types=[
            pltpu.VMEM((CPW * P,), jnp.float32),
            pltpu.VMEM((2, CPW, BX, NY), jnp.float32),
            pltpu.VMEM((3, CAP), jnp.int32),
            pltpu.SemaphoreType.DMA,
            pltpu.SemaphoreType.DMA,
        ],
    )
    def k(ft_hbm, plist_hbm, out_hbm, r01, st, lst_v, sem0, sem_l):
        wid = lax.axis_index("s") * NC + lax.axis_index("c")
        c0 = wid * CPW
        pltpu.sync_copy(ft_hbm.at[c0], r01.at[pl.ds(0, P)])
        pltpu.sync_copy(ft_hbm.at[c0 + 1], r01.at[pl.ds(P, P)])
        iota = lax.iota(jnp.int32, L)
        zero16 = jnp.zeros((L,), jnp.float32)
        zero16i = jnp.zeros((L,), jnp.int32)
        lst_v[0, pl.ds(0, L)] = zero16i
        lst_v[1, pl.ds(0, L)] = zero16i
        lst_v[2, pl.ds(0, L)] = zero16i

        for p in range(2):
            for kk in range(CPW):
                for xr in range(BX):
                    @plsc.parallel_loop(0, NYL, 1, unroll=8)
                    def _(yi, p=p, kk=kk, xr=xr):
                        st[p, kk, xr, pl.ds(yi * L, L)] = zero16

        def unpack(pc):
            return pc & 0xFFFF, (pc >> 16) & 7, pc >> 19

        def dummy_copy():
            return pltpu.make_async_copy(
                st.at[0], out_hbm.at[0, pl.ds(0, CPW), pl.ds(0, BX), :], sem0)

        def dummy_list_copy():
            return pltpu.make_async_copy(
                plist_hbm.at[pl.ds(0, CAP)], lst_v.at[0], sem_l)

        pltpu.async_copy(plist_hbm.at[pl.ds(0, CAP)], lst_v.at[0], sem_l)

        for b in range(B):
            def task(jb, carry):
                t = b * NBX + jb
                p = lax.rem(t, 2)
                s3 = lax.rem(t, 3)
                u3 = lax.rem(t + 1, 3)
                j = t
                cond = jb >= (2 if b == 0 else 0)

                @pl.when(cond)
                def _():
                    dummy_copy().wait()

                pn = lst_v[u3, pl.ds(0, L)][0]
                pnv = (pn + (L - 1)) >> 4

                def undo(v, carry2):
                    m = iota < (pn - v * L)
                    _, cxl, yy = unpack(lst_v[u3, pl.ds(16 + v * L, L)])
                    plsc.store_scatter(st.at[p, 0], [cxl, yy], zero16, mask=m)
                    plsc.store_scatter(st.at[p, 1], [cxl, yy], zero16, mask=m)
                    return carry2

                lax.fori_loop(0, pnv, undo, 0)

                @pl.when(j + 1 < NBINS)
                def _():
                    pltpu.async_copy(
                        plist_hbm.at[pl.ds((j + 1) * CAP, CAP)],
                        lst_v.at[u3], sem_l)

                dummy_list_copy().wait()
                n = lst_v[s3, pl.ds(0, L)][0]
                nv = (n + (L - 1)) >> 4

                def fill(v, carry2):
                    m = iota < (n - v * L)
                    pid, cxl, yy = unpack(lst_v[s3, pl.ds(16 + v * L, L)])
                    v0 = plsc.load_gather(r01, [pid], mask=m)
                    plsc.store_scatter(st.at[p, 0], [cxl, yy], v0, mask=m)
                    v1 = plsc.load_gather(r01, [pid + P], mask=m)
                    plsc.store_scatter(st.at[p, 1], [cxl, yy], v1, mask=m)
                    return carry2

                lax.fori_loop(0, nv, fill, 0)

                x0 = jb * BX
                pltpu.async_copy(
                    st.at[p],
                    out_hbm.at[b, pl.ds(c0, CPW), pl.ds(x0, BX), :], sem0)
                return carry

            lax.fori_loop(0, NBX, task, 0)

        for _ in range(2):
            dummy_copy().wait()

    return k(ft, plist)


def kernel(pillar_features, coords, batch_size, input_shape):
    del batch_size, input_shape
    coords = coords.astype(jnp.int32)
    cy = coords[:, 2]
    cx = coords[:, 3]
    ft = pillar_features.T
    plist = _sc_bin(cy, cx)
    out = _sc_scatter(ft, plist)
    return out.swapaxes(2, 3)
